# baseline (speedup 1.0000x reference)
import functools

import jax
import jax.numpy as jnp
from jax import lax
from jax.experimental import pallas as pl
from jax.experimental.pallas import tpu as pltpu

_LANE = 128


def _round_up(x, m):
    return (x + m - 1) // m * m


def _conv_stats_kernel(x_ref, w_ref, ident_ref, convt_ref, stats_ref,
                       shifted_ref, *, H, W, TH, Hg, Wo, Cin, Coutp, KH, KW,
                       dilation, padding):
    xt = lax.dot_general(x_ref[...], ident_ref[...],
                         (((0,), (0,)), ((), ())),
                         preferred_element_type=jnp.float32)
    xtb = xt.astype(jnp.bfloat16).reshape(H, W, Cin)

    zrow = jnp.zeros((padding, Wo, KW * Cin), jnp.bfloat16)
    shifted_ref[0:padding] = zrow
    shifted_ref[padding + H:padding + H + padding] = zrow
    for kw in range(KW):
        shift = kw * dilation - padding
        lo, hi = kw * Cin, (kw + 1) * Cin
        if shift < 0:
            shifted_ref[pl.ds(padding, H), -shift:Wo, lo:hi] = (
                xtb[:, 0:W + shift, :])
            shifted_ref[pl.ds(padding, H), 0:-shift, lo:hi] = (
                jnp.zeros((H, -shift, Cin), jnp.bfloat16))
        elif shift == 0:
            shifted_ref[pl.ds(padding, H), :, lo:hi] = xtb[:, 0:Wo, :]
        else:
            shifted_ref[pl.ds(padding, H), 0:W - shift, lo:hi] = (
                xtb[:, shift:W, :])
            shifted_ref[pl.ds(padding, H), W - shift:Wo, lo:hi] = (
                jnp.zeros((H, shift, Cin), jnp.bfloat16))

    P = TH * Wo
    halo = dilation * (KH - 1)
    s_acc = jnp.zeros((1, Coutp), jnp.float32)
    ss_acc = jnp.zeros((1, Coutp), jnp.float32)
    for sidx in range(Hg):
        row0 = sidx * TH
        region = shifted_ref[pl.ds(row0, TH + halo), :, :]
        acc = jnp.zeros((P, Coutp), jnp.float32)
        for kh in range(KH):
            lhs = region[kh * dilation:kh * dilation + TH]
            lhs = lhs.reshape(P, KW * Cin)
            w_kh = w_ref[pl.ds(kh * KW * Cin, KW * Cin), :]
            acc = acc + jnp.dot(lhs, w_kh,
                                preferred_element_type=jnp.float32)

        s_acc = s_acc + jnp.sum(acc, axis=0, keepdims=True)
        ss_acc = ss_acc + jnp.sum(acc * acc, axis=0, keepdims=True)

        accb = acc.astype(jnp.bfloat16)
        acct = jnp.transpose(accb)
        convt_ref[:, pl.ds(sidx * P, P)] = acct

    stats_ref[...] = jnp.concatenate([s_acc, ss_acc], axis=0)


def _bn_relu_kernel(stats_ref, gamma_ref, beta_ref, convt_ref, out_ref, *,
                    cnt, eps):
    tot = jnp.sum(stats_ref[...], axis=0)
    mean = tot[0:1, :] / cnt
    var = jnp.maximum(tot[1:2, :] / cnt - mean * mean, 0.0)
    inv = lax.rsqrt(var + eps)
    scale_r = gamma_ref[...] * inv
    shift_r = beta_ref[...] - mean * scale_r
    scale = jnp.transpose(scale_r)
    shift = jnp.transpose(shift_r)

    y = convt_ref[...].astype(jnp.float32) * scale + shift
    out_ref[...] = jnp.maximum(y, 0.0).astype(out_ref.dtype)


def kernel(x_nchw, weight_oihw, gamma, beta):
    padding, dilation, eps = 2, 2, 1e-5
    N, Cin, H, W = x_nchw.shape
    Cout, _, KH, KW = weight_oihw.shape

    Ho = H + 2 * padding - dilation * (KH - 1)
    Wo = W + 2 * padding - dilation * (KW - 1)
    Hp, Wp = H + 2 * padding, W + 2 * padding

    Coutp = _round_up(Cout, _LANE)

    x3 = x_nchw.reshape(N, Cin, H * W)
    ident = jnp.eye(Cin, dtype=jnp.float32)

    w = jnp.transpose(weight_oihw, (2, 3, 1, 0))
    w = jnp.pad(w, ((0, 0), (0, 0), (0, 0), (0, Coutp - Cout)))
    w2d = w.reshape(KH * KW * Cin, Coutp).astype(jnp.bfloat16)

    TH = 8
    Hg = Ho // TH
    P = TH * Wo

    conv_kernel = functools.partial(
        _conv_stats_kernel, H=H, W=W, TH=TH, Hg=Hg, Wo=Wo, Cin=Cin,
        Coutp=Coutp, KH=KH, KW=KW, dilation=dilation, padding=padding)

    convt, stats = pl.pallas_call(
        conv_kernel,
        out_shape=(jax.ShapeDtypeStruct((N, Coutp, Ho * Wo), jnp.bfloat16),
                   jax.ShapeDtypeStruct((N, 2, Coutp), jnp.float32)),
        grid=(N,),
        in_specs=[
            pl.BlockSpec((None, Cin, H * W), lambda n: (n, 0, 0)),
            pl.BlockSpec((KH * KW * Cin, Coutp), lambda n: (0, 0)),
            pl.BlockSpec((Cin, Cin), lambda n: (0, 0)),
        ],
        out_specs=(
            pl.BlockSpec((None, Coutp, Ho * Wo), lambda n: (n, 0, 0)),
            pl.BlockSpec((None, 2, Coutp), lambda n: (n, 0, 0)),
        ),
        scratch_shapes=[pltpu.VMEM((H + 2 * padding, Wo, KW * Cin),
                                   jnp.bfloat16)],
        compiler_params=pltpu.CompilerParams(
            dimension_semantics=("parallel",),
            vmem_limit_bytes=32 * 1024 * 1024),
    )(x3, w2d, ident)

    gamma_r = jnp.pad(gamma.astype(jnp.float32), (0, Coutp - Cout))
    beta_r = jnp.pad(beta.astype(jnp.float32), (0, Coutp - Cout))
    gamma_r = gamma_r.reshape(1, Coutp)
    beta_r = beta_r.reshape(1, Coutp)

    bn_kernel = functools.partial(_bn_relu_kernel,
                                  cnt=float(N * Ho * Wo), eps=eps)

    out = pl.pallas_call(
        bn_kernel,
        out_shape=jax.ShapeDtypeStruct((N, Coutp, Ho * Wo), jnp.float32),
        grid=(N,),
        in_specs=[
            pl.BlockSpec((N, 2, Coutp), lambda n: (0, 0, 0)),
            pl.BlockSpec((1, Coutp), lambda n: (0, 0)),
            pl.BlockSpec((1, Coutp), lambda n: (0, 0)),
            pl.BlockSpec((None, Coutp, Ho * Wo), lambda n: (n, 0, 0)),
        ],
        out_specs=pl.BlockSpec((None, Coutp, Ho * Wo), lambda n: (n, 0, 0)),
        compiler_params=pltpu.CompilerParams(
            dimension_semantics=("parallel",)),
    )(stats, gamma_r, beta_r, convt)

    return out[:, :Cout, :].reshape(N, Cout, Ho, Wo)


# speedup vs baseline: 1.9180x; 1.9180x over previous
import functools

import jax
import jax.numpy as jnp
from jax import lax
from jax.experimental import pallas as pl
from jax.experimental.pallas import tpu as pltpu

_LANE = 128


def _round_up(x, m):
    return (x + m - 1) // m * m


def _conv_stats_kernel(x_ref, w_ref, conv_ref, stats_ref, shifted_ref, *,
                       H, W, TH, Hg, Wo, Cin, Coutp, KH, KW, dilation,
                       padding):
    xb = x_ref[...].astype(jnp.bfloat16)

    zrow = jnp.zeros((padding, Wo, KW * Cin), jnp.bfloat16)
    shifted_ref[0:padding] = zrow
    shifted_ref[padding + H:padding + H + padding] = zrow
    for kw in range(KW):
        shift = kw * dilation - padding
        lo, hi = kw * Cin, (kw + 1) * Cin
        if shift < 0:
            shifted_ref[pl.ds(padding, H), -shift:Wo, lo:hi] = (
                xb[:, 0:W + shift, :])
            shifted_ref[pl.ds(padding, H), 0:-shift, lo:hi] = (
                jnp.zeros((H, -shift, Cin), jnp.bfloat16))
        elif shift == 0:
            shifted_ref[pl.ds(padding, H), :, lo:hi] = xb[:, 0:Wo, :]
        else:
            shifted_ref[pl.ds(padding, H), 0:W - shift, lo:hi] = (
                xb[:, shift:W, :])
            shifted_ref[pl.ds(padding, H), W - shift:Wo, lo:hi] = (
                jnp.zeros((H, shift, Cin), jnp.bfloat16))

    P = TH * Wo
    halo = dilation * (KH - 1)
    s_acc = jnp.zeros((1, Coutp), jnp.float32)
    ss_acc = jnp.zeros((1, Coutp), jnp.float32)
    for sidx in range(Hg):
        row0 = sidx * TH
        region = shifted_ref[pl.ds(row0, TH + halo), :, :]
        acc = jnp.zeros((P, Coutp), jnp.float32)
        for kh in range(KH):
            lhs = region[kh * dilation:kh * dilation + TH]
            lhs = lhs.reshape(P, KW * Cin)
            w_kh = w_ref[pl.ds(kh * KW * Cin, KW * Cin), :]
            acc = acc + jnp.dot(lhs, w_kh,
                                preferred_element_type=jnp.float32)

        s_acc = s_acc + jnp.sum(acc, axis=0, keepdims=True)
        ss_acc = ss_acc + jnp.sum(acc * acc, axis=0, keepdims=True)

        conv_ref[pl.ds(sidx * P, P), :] = acc.astype(jnp.bfloat16)

    stats_ref[...] = jnp.concatenate([s_acc, ss_acc], axis=0)


def _bn_relu_kernel(stats_ref, gamma_ref, beta_ref, conv_ref, out_ref, *,
                    cnt, eps):
    tot = jnp.sum(stats_ref[...], axis=0)
    mean = tot[0:1, :] / cnt
    var = jnp.maximum(tot[1:2, :] / cnt - mean * mean, 0.0)
    inv = lax.rsqrt(var + eps)
    scale = gamma_ref[...] * inv
    shift = beta_ref[...] - mean * scale

    y = conv_ref[...].astype(jnp.float32) * scale + shift
    out_ref[...] = jnp.maximum(y, 0.0).astype(out_ref.dtype)


def kernel(x_nchw, weight_oihw, gamma, beta):
    padding, dilation, eps = 2, 2, 1e-5
    N, Cin, H, W = x_nchw.shape
    Cout, _, KH, KW = weight_oihw.shape

    Ho = H + 2 * padding - dilation * (KH - 1)
    Wo = W + 2 * padding - dilation * (KW - 1)

    Coutp = _round_up(Cout, _LANE)

    x_nhwc = jnp.transpose(x_nchw, (0, 2, 3, 1))

    w = jnp.transpose(weight_oihw, (2, 3, 1, 0))
    w = jnp.pad(w, ((0, 0), (0, 0), (0, 0), (0, Coutp - Cout)))
    w2d = w.reshape(KH * KW * Cin, Coutp).astype(jnp.bfloat16)

    TH = 8
    Hg = Ho // TH
    P = TH * Wo

    conv_kernel = functools.partial(
        _conv_stats_kernel, H=H, W=W, TH=TH, Hg=Hg, Wo=Wo, Cin=Cin,
        Coutp=Coutp, KH=KH, KW=KW, dilation=dilation, padding=padding)

    conv, stats = pl.pallas_call(
        conv_kernel,
        out_shape=(jax.ShapeDtypeStruct((N, Ho * Wo, Coutp), jnp.bfloat16),
                   jax.ShapeDtypeStruct((N, 2, Coutp), jnp.float32)),
        grid=(N,),
        in_specs=[
            pl.BlockSpec((None, H, W, Cin), lambda n: (n, 0, 0, 0)),
            pl.BlockSpec((KH * KW * Cin, Coutp), lambda n: (0, 0)),
        ],
        out_specs=(
            pl.BlockSpec((None, Ho * Wo, Coutp), lambda n: (n, 0, 0)),
            pl.BlockSpec((None, 2, Coutp), lambda n: (n, 0, 0)),
        ),
        scratch_shapes=[pltpu.VMEM((H + 2 * padding, Wo, KW * Cin),
                                   jnp.bfloat16)],
        compiler_params=pltpu.CompilerParams(
            dimension_semantics=("parallel",),
            vmem_limit_bytes=32 * 1024 * 1024),
    )(x_nhwc, w2d)

    gamma_r = jnp.pad(gamma.astype(jnp.float32), (0, Coutp - Cout))
    beta_r = jnp.pad(beta.astype(jnp.float32), (0, Coutp - Cout))
    gamma_r = gamma_r.reshape(1, Coutp)
    beta_r = beta_r.reshape(1, Coutp)

    bn_kernel = functools.partial(_bn_relu_kernel,
                                  cnt=float(N * Ho * Wo), eps=eps)

    out = pl.pallas_call(
        bn_kernel,
        out_shape=jax.ShapeDtypeStruct((N, Ho * Wo, Coutp), jnp.float32),
        grid=(N,),
        in_specs=[
            pl.BlockSpec((N, 2, Coutp), lambda n: (0, 0, 0)),
            pl.BlockSpec((1, Coutp), lambda n: (0, 0)),
            pl.BlockSpec((1, Coutp), lambda n: (0, 0)),
            pl.BlockSpec((None, Ho * Wo, Coutp), lambda n: (n, 0, 0)),
        ],
        out_specs=pl.BlockSpec((None, Ho * Wo, Coutp), lambda n: (n, 0, 0)),
        compiler_params=pltpu.CompilerParams(
            dimension_semantics=("parallel",)),
    )(stats, gamma_r, beta_r, conv)

    out_nhwc = out[:, :, :Cout].reshape(N, Ho, Wo, Cout)
    return jnp.transpose(out_nhwc, (0, 3, 1, 2))
